# baseline (speedup 1.0000x reference)
import functools

import jax
import jax.numpy as jnp
from jax.experimental import pallas as pl

NEG_INF = -1000000000.0
_BR = 64


def _softmax3_block(x_ref, o_ref, *, n, block_rows):
    w = 3 * n
    x = x_ref[...]
    lane = jax.lax.broadcasted_iota(jnp.int32, x.shape, 1)
    row = jax.lax.broadcasted_iota(jnp.int32, x.shape, 0) + (
        pl.program_id(0) * block_rows
    )
    mod3 = lane % 3
    x = jnp.where(lane == w - 2, NEG_INF, x)
    x = jnp.where((row == n - 1) & (mod3 == 0), NEG_INF, x)
    u = jnp.roll(x, -1, axis=1)
    v = jnp.roll(x, -2, axis=1)
    p = jnp.roll(x, 1, axis=1)
    q = jnp.roll(x, 2, axis=1)
    o1 = jnp.where(mod3 == 0, u, jnp.where(mod3 == 1, p, q))
    o2 = jnp.where(mod3 == 0, v, jnp.where(mod3 == 1, u, p))
    o_ref[...] = 1.0 / (1.0 + jnp.exp(o1 - x) + jnp.exp(o2 - x))


def kernel(log_edge_flows):
    n = log_edge_flows.shape[0]
    w = 3 * n
    x2d = log_edge_flows.reshape(n, w)
    grid = pl.cdiv(n, _BR)
    out = pl.pallas_call(
        functools.partial(_softmax3_block, n=n, block_rows=_BR),
        grid=(grid,),
        in_specs=[pl.BlockSpec((_BR, w), lambda i: (i, 0))],
        out_specs=pl.BlockSpec((_BR, w), lambda i: (i, 0)),
        out_shape=jax.ShapeDtypeStruct((n, w), jnp.float32),
    )(x2d)
    return out.reshape(n, n, 3)
